# Initial kernel scaffold; baseline (speedup 1.0000x reference)
import jax
import jax.numpy as jnp
from jax.experimental import pallas as pl


def kernel(x, edge_index, Ws0, Wn0, b0, Ws1, Wn1, b1, Ws2, Wn2, b2):
    raise NotImplementedError("write your pallas kernel here")


# speedup vs baseline: 4.6199x; 4.6199x over previous
import functools

import jax
import jax.numpy as jnp
from jax import lax
from jax.experimental import pallas as pl
from jax.experimental.pallas import tpu as pltpu
from jax.experimental.pallas import tpu_sc as plsc

_C = 125
_NSUB = 16
_NCORE = 2


def _tc_mm0(x, Ws, Wn, b, *, bn=1000):
    n = x.shape[0]
    nb = n // bn

    def body(x_ref, ws_ref, wn_ref, b_ref, xs_ref, y2_ref):
        h = x_ref[...]
        xs_ref[...] = (
            jnp.dot(h, ws_ref[...], preferred_element_type=jnp.float32)
            + b_ref[...]
        )
        y2_ref[...] = jnp.dot(h, wn_ref[...], preferred_element_type=jnp.float32)

    return pl.pallas_call(
        body,
        grid=(nb, 2),
        in_specs=[
            pl.BlockSpec((bn, 256), lambda i, j: (i, 0)),
            pl.BlockSpec((256, 128), lambda i, j: (0, j)),
            pl.BlockSpec((256, 128), lambda i, j: (0, j)),
            pl.BlockSpec((1, 128), lambda i, j: (0, j)),
        ],
        out_specs=[
            pl.BlockSpec((bn, 128), lambda i, j: (i, j)),
            pl.BlockSpec((bn, 128), lambda i, j: (j * nb + i, 0)),
        ],
        out_shape=[
            jax.ShapeDtypeStruct((n, 256), jnp.float32),
            jax.ShapeDtypeStruct((2 * n, 128), jnp.float32),
        ],
    )(x, Ws, Wn, b.reshape(1, 256))


def _tc_combine_mm(xs_prev, agg, deg, Ws, Wn, b, *, bn=1000):
    n = xs_prev.shape[0]
    nb = n // bn

    def body(xs_ref, a0_ref, a1_ref, d_ref, ws_ref, wn_ref, b_ref, o_ref, y2_ref):
        inv = 1.0 / jnp.maximum(d_ref[:, 0:1], 1.0)
        agg_full = jnp.concatenate([a0_ref[...], a1_ref[...]], axis=1)
        h = jnp.maximum(xs_ref[...] + agg_full * inv, 0.0)
        o_ref[...] = (
            jnp.dot(h, ws_ref[...], preferred_element_type=jnp.float32)
            + b_ref[...]
        )
        y2_ref[...] = jnp.dot(h, wn_ref[...], preferred_element_type=jnp.float32)

    return pl.pallas_call(
        body,
        grid=(nb, 2),
        in_specs=[
            pl.BlockSpec((bn, 256), lambda i, j: (i, 0)),
            pl.BlockSpec((bn, 128), lambda i, j: (i, 0)),
            pl.BlockSpec((bn, 128), lambda i, j: (nb + i, 0)),
            pl.BlockSpec((bn, 128), lambda i, j: (i, 0)),
            pl.BlockSpec((256, 128), lambda i, j: (0, j)),
            pl.BlockSpec((256, 128), lambda i, j: (0, j)),
            pl.BlockSpec((1, 128), lambda i, j: (0, j)),
        ],
        out_specs=[
            pl.BlockSpec((bn, 128), lambda i, j: (i, j)),
            pl.BlockSpec((bn, 128), lambda i, j: (j * nb + i, 0)),
        ],
        out_shape=[
            jax.ShapeDtypeStruct((n, 256), jnp.float32),
            jax.ShapeDtypeStruct((2 * n, 128), jnp.float32),
        ],
    )(xs_prev, agg, agg, deg, Ws, Wn, b.reshape(1, 256))


def _tc_final(xs_prev, agg, deg, *, bn=1000):
    n = xs_prev.shape[0]
    nb = n // bn

    def body(xs_ref, a0_ref, a1_ref, d_ref, o_ref):
        inv = 1.0 / jnp.maximum(d_ref[:, 0:1], 1.0)
        agg_full = jnp.concatenate([a0_ref[...], a1_ref[...]], axis=1)
        o_ref[...] = xs_ref[...] + agg_full * inv

    return pl.pallas_call(
        body,
        grid=(nb,),
        in_specs=[
            pl.BlockSpec((bn, 256), lambda i: (i, 0)),
            pl.BlockSpec((bn, 128), lambda i: (i, 0)),
            pl.BlockSpec((bn, 128), lambda i: (nb + i, 0)),
            pl.BlockSpec((bn, 128), lambda i: (i, 0)),
        ],
        out_specs=pl.BlockSpec((bn, 256), lambda i: (i, 0)),
        out_shape=jax.ShapeDtypeStruct((n, 256), jnp.float32),
    )(xs_prev, agg, agg, deg)


_B = 8


@functools.partial(jax.jit, static_argnames=("n", "e"))
def _sc_segment_sum(y2, src_pc, dst2, z128, *, n, e):
    nch = e // _C
    per_tile = nch // _NSUB
    rpt = (n // _NSUB) // 8 * 8
    rem = n - _NSUB * rpt

    mesh = plsc.VectorSubcoreMesh(core_axis_name="c", subcore_axis_name="s")

    def body(y2_h, src_h, dst_h, z128_h, agg_o, src_v, dst_v, rows_v, acc, sem):
        c = lax.axis_index("c")
        s = lax.axis_index("s")
        r0 = s * rpt
        rr = _NSUB * rpt

        pltpu.sync_copy(z128_h.at[pl.ds(r0, rpt)], acc.at[pl.ds(r0, rpt)])

        @pl.when(s == _NSUB - 1)
        def _():
            pltpu.sync_copy(z128_h.at[pl.ds(rr, rem)], acc.at[pl.ds(rr, rem)])

        plsc.subcore_barrier()

        def batch(bi, carry):
            base = s * per_tile + bi * _B
            pltpu.sync_copy(src_h.at[pl.ds(c * nch + base, _B)], src_v)
            pltpu.sync_copy(dst_h.at[pl.ds(base, _B)], dst_v)

            def step(k, carry2):
                pltpu.async_copy(y2_h.at[src_v.at[k]], rows_v, sem).wait()
                pltpu.sync_copy(rows_v, acc.at[dst_v.at[k]], add=True)
                return carry2

            return lax.fori_loop(0, _B, step, carry)

        lax.fori_loop(0, per_tile // _B, batch, 0)
        plsc.subcore_barrier()

        pltpu.sync_copy(acc.at[pl.ds(r0, rpt)],
                        agg_o.at[pl.ds(c * n + r0, rpt)])

        @pl.when(s == _NSUB - 1)
        def _():
            pltpu.sync_copy(acc.at[pl.ds(rr, rem)],
                            agg_o.at[pl.ds(c * n + rr, rem)])

    fn = pl.kernel(
        body,
        out_type=jax.ShapeDtypeStruct((2 * n, 128), jnp.float32),
        mesh=mesh,
        scratch_types=[
            pltpu.VMEM((_B, _C), jnp.int32),
            pltpu.VMEM((_B, _C), jnp.int32),
            pltpu.VMEM((_C, 128), jnp.float32),
            pltpu.VMEM_SHARED((n, 128), jnp.float32),
            pltpu.SemaphoreType.DMA,
        ],
    )
    return fn(y2, src_pc, dst2, z128)


@functools.partial(jax.jit, static_argnames=("n", "e"))
def _sc_degree(dst2, ones128, z128, *, n, e):
    nch = e // _C
    per_tile = nch // _NSUB
    rpt = (n // _NSUB) // 8 * 8
    rem = n - _NSUB * rpt

    mesh = plsc.VectorSubcoreMesh(core_axis_name="c", subcore_axis_name="s")

    def body(dst_h, ones_h, z8_h, deg_o, dst_v, ones_v, accd, sem):
        c = lax.axis_index("c")
        s = lax.axis_index("s")

        @pl.when(c == 0)
        def _():
            r0 = s * rpt
            rr = _NSUB * rpt
            pltpu.sync_copy(z8_h.at[pl.ds(r0, rpt)], accd.at[pl.ds(r0, rpt)])
            pltpu.sync_copy(ones_h, ones_v)

            @pl.when(s == _NSUB - 1)
            def _():
                pltpu.sync_copy(z8_h.at[pl.ds(rr, rem)],
                                accd.at[pl.ds(rr, rem)])

            plsc.subcore_barrier()

            def batch(bi, carry):
                base = s * per_tile + bi * _B
                pltpu.sync_copy(dst_h.at[pl.ds(base, _B)], dst_v)

                def step(k, carry2):
                    pltpu.sync_copy(ones_v, accd.at[dst_v.at[k]], add=True)
                    return carry2

                return lax.fori_loop(0, _B, step, carry)

            lax.fori_loop(0, per_tile // _B, batch, 0)
            plsc.subcore_barrier()
            pltpu.sync_copy(accd.at[pl.ds(r0, rpt)], deg_o.at[pl.ds(r0, rpt)])

            @pl.when(s == _NSUB - 1)
            def _():
                pltpu.sync_copy(accd.at[pl.ds(rr, rem)],
                                deg_o.at[pl.ds(rr, rem)])

    fn = pl.kernel(
        body,
        out_type=jax.ShapeDtypeStruct((n, 128), jnp.float32),
        mesh=mesh,
        scratch_types=[
            pltpu.VMEM((_B, _C), jnp.int32),
            pltpu.VMEM((_C, 128), jnp.float32),
            pltpu.VMEM_SHARED((n, 128), jnp.float32),
            pltpu.SemaphoreType.DMA,
        ],
    )
    return fn(dst2, ones128, z128)


def kernel(x, edge_index, Ws0, Wn0, b0, Ws1, Wn1, b1, Ws2, Wn2, b2):
    n = x.shape[0]
    e = edge_index.shape[1]
    src = edge_index[0].astype(jnp.int32)
    dst = edge_index[1].astype(jnp.int32)

    src_pc = jnp.concatenate([src, src + n]).reshape(2 * (e // _C), _C)
    dst2 = dst.reshape(e // _C, _C)
    ones128 = jnp.ones((_C, 128), jnp.float32)
    z128 = jnp.zeros((n, 128), jnp.float32)

    deg = _sc_degree(dst2, ones128, z128, n=n, e=e)
    xs0, y0 = _tc_mm0(x, Ws0, Wn0, b0)
    agg0 = _sc_segment_sum(y0, src_pc, dst2, z128, n=n, e=e)
    xs1, y1 = _tc_combine_mm(xs0, agg0, deg, Ws1, Wn1, b1)
    agg1 = _sc_segment_sum(y1, src_pc, dst2, z128, n=n, e=e)
    xs2, y2 = _tc_combine_mm(xs1, agg1, deg, Ws2, Wn2, b2)
    agg2 = _sc_segment_sum(y2, src_pc, dst2, z128, n=n, e=e)
    return _tc_final(xs2, agg2, deg)


# speedup vs baseline: 5.9524x; 1.2884x over previous
import functools

import jax
import jax.numpy as jnp
from jax import lax
from jax.experimental import pallas as pl
from jax.experimental.pallas import tpu as pltpu
from jax.experimental.pallas import tpu_sc as plsc

_C = 125
_NSUB = 16
_NCORE = 2


def _tc_mm0(x, Ws, Wn, b, *, bn=1000):
    n = x.shape[0]
    nb = n // bn

    def body(x_ref, ws_ref, wn_ref, b_ref, xs_ref, y2_ref):
        h = x_ref[...]
        xs_ref[...] = (
            jnp.dot(h, ws_ref[...], preferred_element_type=jnp.float32)
            + b_ref[...]
        )
        y2_ref[...] = jnp.dot(h, wn_ref[...], preferred_element_type=jnp.float32)

    return pl.pallas_call(
        body,
        grid=(nb, 2),
        in_specs=[
            pl.BlockSpec((bn, 256), lambda i, j: (i, 0)),
            pl.BlockSpec((256, 128), lambda i, j: (0, j)),
            pl.BlockSpec((256, 128), lambda i, j: (0, j)),
            pl.BlockSpec((1, 128), lambda i, j: (0, j)),
        ],
        out_specs=[
            pl.BlockSpec((bn, 128), lambda i, j: (i, j)),
            pl.BlockSpec((bn, 128), lambda i, j: (j * nb + i, 0)),
        ],
        out_shape=[
            jax.ShapeDtypeStruct((n, 256), jnp.float32),
            jax.ShapeDtypeStruct((2 * n, 128), jnp.float32),
        ],
    )(x, Ws, Wn, b.reshape(1, 256))


def _tc_combine_mm(xs_prev, agg, deg, Ws, Wn, b, *, bn=1000):
    n = xs_prev.shape[0]
    nb = n // bn

    def body(xs_ref, a0_ref, a1_ref, d0_ref, d1_ref, ws_ref, wn_ref, b_ref,
             o_ref, y2_ref):
        inv = 1.0 / jnp.maximum(d0_ref[:, 0:1] + d1_ref[:, 0:1], 1.0)
        agg_full = jnp.concatenate([a0_ref[...], a1_ref[...]], axis=1)
        h = jnp.maximum(xs_ref[...] + agg_full * inv, 0.0)
        o_ref[...] = (
            jnp.dot(h, ws_ref[...], preferred_element_type=jnp.float32)
            + b_ref[...]
        )
        y2_ref[...] = jnp.dot(h, wn_ref[...], preferred_element_type=jnp.float32)

    return pl.pallas_call(
        body,
        grid=(nb, 2),
        in_specs=[
            pl.BlockSpec((bn, 256), lambda i, j: (i, 0)),
            pl.BlockSpec((bn, 128), lambda i, j: (i, 0)),
            pl.BlockSpec((bn, 128), lambda i, j: (nb + i, 0)),
            pl.BlockSpec((bn, 128), lambda i, j: (i, 0)),
            pl.BlockSpec((bn, 128), lambda i, j: (nb + i, 0)),
            pl.BlockSpec((256, 128), lambda i, j: (0, j)),
            pl.BlockSpec((256, 128), lambda i, j: (0, j)),
            pl.BlockSpec((1, 128), lambda i, j: (0, j)),
        ],
        out_specs=[
            pl.BlockSpec((bn, 128), lambda i, j: (i, j)),
            pl.BlockSpec((bn, 128), lambda i, j: (j * nb + i, 0)),
        ],
        out_shape=[
            jax.ShapeDtypeStruct((n, 256), jnp.float32),
            jax.ShapeDtypeStruct((2 * n, 128), jnp.float32),
        ],
    )(xs_prev, agg, agg, deg, deg, Ws, Wn, b.reshape(1, 256))


def _tc_final(xs_prev, agg, deg, *, bn=1000):
    n = xs_prev.shape[0]
    nb = n // bn

    def body(xs_ref, a0_ref, a1_ref, d0_ref, d1_ref, o_ref):
        inv = 1.0 / jnp.maximum(d0_ref[:, 0:1] + d1_ref[:, 0:1], 1.0)
        agg_full = jnp.concatenate([a0_ref[...], a1_ref[...]], axis=1)
        o_ref[...] = xs_ref[...] + agg_full * inv

    return pl.pallas_call(
        body,
        grid=(nb,),
        in_specs=[
            pl.BlockSpec((bn, 256), lambda i: (i, 0)),
            pl.BlockSpec((bn, 128), lambda i: (i, 0)),
            pl.BlockSpec((bn, 128), lambda i: (nb + i, 0)),
            pl.BlockSpec((bn, 128), lambda i: (i, 0)),
            pl.BlockSpec((bn, 128), lambda i: (nb + i, 0)),
        ],
        out_specs=pl.BlockSpec((bn, 256), lambda i: (i, 0)),
        out_shape=jax.ShapeDtypeStruct((n, 256), jnp.float32),
    )(xs_prev, agg, agg, deg, deg)


_B = 16


@functools.partial(jax.jit, static_argnames=("n", "e"))
def _sc_segment_sum(y2, src_pc, dst2, z128, *, n, e):
    nch = e // _C
    per_tile = nch // _NSUB
    nbatch = per_tile // _B
    rpt = (n // _NSUB) // 8 * 8
    rem = n - _NSUB * rpt

    mesh = plsc.VectorSubcoreMesh(core_axis_name="c", subcore_axis_name="s")

    def body(y2_h, src_h, dst_h, z128_h, agg_o,
             src_v, dst_v, rows_a, rows_b, acc, semg, sems_a, sems_b):
        c = lax.axis_index("c")
        s = lax.axis_index("s")
        r0 = s * rpt
        rr = _NSUB * rpt

        pltpu.sync_copy(z128_h.at[pl.ds(r0, rpt)], acc.at[pl.ds(r0, rpt)])

        @pl.when(s == _NSUB - 1)
        def _():
            pltpu.sync_copy(z128_h.at[pl.ds(rr, rem)], acc.at[pl.ds(rr, rem)])

        plsc.subcore_barrier()

        rows = (rows_a, rows_b)
        sems = (sems_a, sems_b)

        def batch(bi, carry):
            base = s * per_tile + bi * _B
            pltpu.sync_copy(src_h.at[pl.ds(c * nch + base, _B)], src_v)
            pltpu.sync_copy(dst_h.at[pl.ds(base, _B)], dst_v)
            for kk in range(_B):
                b = kk & 1
                if kk >= 2:
                    pltpu.make_async_copy(
                        rows[b], acc.at[dst_v.at[kk - 2]], sems[b]).wait()
                pltpu.async_copy(y2_h.at[src_v.at[kk]], rows[b], semg).wait()
                pltpu.async_copy(rows[b], acc.at[dst_v.at[kk]], sems[b],
                                 add=True)
            pltpu.make_async_copy(rows[0], acc.at[dst_v.at[_B - 2]],
                                  sems[0]).wait()
            pltpu.make_async_copy(rows[1], acc.at[dst_v.at[_B - 1]],
                                  sems[1]).wait()
            return carry

        lax.fori_loop(0, nbatch, batch, 0)
        plsc.subcore_barrier()

        pltpu.sync_copy(acc.at[pl.ds(r0, rpt)],
                        agg_o.at[pl.ds(c * n + r0, rpt)])

        @pl.when(s == _NSUB - 1)
        def _():
            pltpu.sync_copy(acc.at[pl.ds(rr, rem)],
                            agg_o.at[pl.ds(c * n + rr, rem)])

    fn = pl.kernel(
        body,
        out_type=jax.ShapeDtypeStruct((2 * n, 128), jnp.float32),
        mesh=mesh,
        scratch_types=[
            pltpu.VMEM((_B, _C), jnp.int32),
            pltpu.VMEM((_B, _C), jnp.int32),
            pltpu.VMEM((_C, 128), jnp.float32),
            pltpu.VMEM((_C, 128), jnp.float32),
            pltpu.VMEM_SHARED((n, 128), jnp.float32),
            pltpu.SemaphoreType.DMA,
            pltpu.SemaphoreType.DMA,
            pltpu.SemaphoreType.DMA,
        ],
    )
    return fn(y2, src_pc, dst2, z128)


@functools.partial(jax.jit, static_argnames=("n", "e"))
def _sc_degree(dst2, ones128, z128, *, n, e):
    nch = e // _C
    half = nch // 2
    per_tile = half // _NSUB
    db = 8
    nbatch = per_tile // db
    rpt = (n // _NSUB) // 8 * 8
    rem = n - _NSUB * rpt

    mesh = plsc.VectorSubcoreMesh(core_axis_name="c", subcore_axis_name="s")

    def body(dst_h, ones_h, z128_h, deg_o, dst_v, ones_v, accd, sem):
        c = lax.axis_index("c")
        s = lax.axis_index("s")
        r0 = s * rpt
        rr = _NSUB * rpt

        pltpu.sync_copy(z128_h.at[pl.ds(r0, rpt)], accd.at[pl.ds(r0, rpt)])
        pltpu.sync_copy(ones_h, ones_v)

        @pl.when(s == _NSUB - 1)
        def _():
            pltpu.sync_copy(z128_h.at[pl.ds(rr, rem)], accd.at[pl.ds(rr, rem)])

        plsc.subcore_barrier()

        def batch(bi, carry):
            base = c * half + s * per_tile + bi * db
            pltpu.sync_copy(dst_h.at[pl.ds(base, db)], dst_v)
            for kk in range(db):
                pltpu.async_copy(ones_v, accd.at[dst_v.at[kk]], sem, add=True)
            for kk in range(db):
                pltpu.make_async_copy(ones_v, accd.at[dst_v.at[kk]],
                                      sem).wait()
            return carry

        lax.fori_loop(0, nbatch, batch, 0)
        plsc.subcore_barrier()
        pltpu.sync_copy(accd.at[pl.ds(r0, rpt)],
                        deg_o.at[pl.ds(c * n + r0, rpt)])

        @pl.when(s == _NSUB - 1)
        def _():
            pltpu.sync_copy(accd.at[pl.ds(rr, rem)],
                            deg_o.at[pl.ds(c * n + rr, rem)])

    fn = pl.kernel(
        body,
        out_type=jax.ShapeDtypeStruct((2 * n, 128), jnp.float32),
        mesh=mesh,
        scratch_types=[
            pltpu.VMEM((8, _C), jnp.int32),
            pltpu.VMEM((_C, 128), jnp.float32),
            pltpu.VMEM_SHARED((n, 128), jnp.float32),
            pltpu.SemaphoreType.DMA,
        ],
    )
    return fn(dst2, ones128, z128)


def kernel(x, edge_index, Ws0, Wn0, b0, Ws1, Wn1, b1, Ws2, Wn2, b2):
    n = x.shape[0]
    e = edge_index.shape[1]
    src = edge_index[0].astype(jnp.int32)
    dst = edge_index[1].astype(jnp.int32)

    src_pc = jnp.concatenate([src, src + n]).reshape(2 * (e // _C), _C)
    dst2 = dst.reshape(e // _C, _C)
    ones128 = jnp.ones((_C, 128), jnp.float32)
    z128 = jnp.zeros((n, 128), jnp.float32)

    deg = _sc_degree(dst2, ones128, z128, n=n, e=e)
    xs0, y0 = _tc_mm0(x, Ws0, Wn0, b0)
    agg0 = _sc_segment_sum(y0, src_pc, dst2, z128, n=n, e=e)
    xs1, y1 = _tc_combine_mm(xs0, agg0, deg, Ws1, Wn1, b1)
    agg1 = _sc_segment_sum(y1, src_pc, dst2, z128, n=n, e=e)
    xs2, y2 = _tc_combine_mm(xs1, agg1, deg, Ws2, Wn2, b2)
    agg2 = _sc_segment_sum(y2, src_pc, dst2, z128, n=n, e=e)
    return _tc_final(xs2, agg2, deg)


# speedup vs baseline: 6.9773x; 1.1722x over previous
import functools

import jax
import jax.numpy as jnp
from jax import lax
from jax.experimental import pallas as pl
from jax.experimental.pallas import tpu as pltpu
from jax.experimental.pallas import tpu_sc as plsc

_C = 125
_NSUB = 16
_NCORE = 2


def _tc_mm0(x, Ws, Wn, b, *, bn=1000):
    n = x.shape[0]
    nb = n // bn

    def body(x_ref, ws_ref, wn_ref, b_ref, xs_ref, y2_ref):
        h = x_ref[...]
        xs_ref[...] = (
            jnp.dot(h, ws_ref[...], preferred_element_type=jnp.float32)
            + b_ref[...]
        )
        y2_ref[...] = jnp.dot(h, wn_ref[...], preferred_element_type=jnp.float32)

    return pl.pallas_call(
        body,
        grid=(nb, 2),
        in_specs=[
            pl.BlockSpec((bn, 256), lambda i, j: (i, 0)),
            pl.BlockSpec((256, 128), lambda i, j: (0, j)),
            pl.BlockSpec((256, 128), lambda i, j: (0, j)),
            pl.BlockSpec((1, 128), lambda i, j: (0, j)),
        ],
        out_specs=[
            pl.BlockSpec((bn, 128), lambda i, j: (i, j)),
            pl.BlockSpec((bn, 128), lambda i, j: (j * nb + i, 0)),
        ],
        out_shape=[
            jax.ShapeDtypeStruct((n, 256), jnp.float32),
            jax.ShapeDtypeStruct((2 * n, 128), jnp.float32),
        ],
    )(x, Ws, Wn, b.reshape(1, 256))


def _tc_combine_mm(xs_prev, agg, deg, Ws, Wn, b, *, bn=1000):
    n = xs_prev.shape[0]
    nb = n // bn

    def body(xs_ref, a0_ref, a1_ref, d0_ref, d1_ref, ws_ref, wn_ref, b_ref,
             o_ref, y2_ref):
        inv = 1.0 / jnp.maximum(d0_ref[:, 0:1] + d1_ref[:, 0:1], 1.0)
        agg_full = jnp.concatenate([a0_ref[...], a1_ref[...]], axis=1)
        h = jnp.maximum(xs_ref[...] + agg_full * inv, 0.0)
        o_ref[...] = (
            jnp.dot(h, ws_ref[...], preferred_element_type=jnp.float32)
            + b_ref[...]
        )
        y2_ref[...] = jnp.dot(h, wn_ref[...], preferred_element_type=jnp.float32)

    return pl.pallas_call(
        body,
        grid=(nb, 2),
        in_specs=[
            pl.BlockSpec((bn, 256), lambda i, j: (i, 0)),
            pl.BlockSpec((bn, 128), lambda i, j: (i, 0)),
            pl.BlockSpec((bn, 128), lambda i, j: (nb + i, 0)),
            pl.BlockSpec((bn, 128), lambda i, j: (i, 0)),
            pl.BlockSpec((bn, 128), lambda i, j: (nb + i, 0)),
            pl.BlockSpec((256, 128), lambda i, j: (0, j)),
            pl.BlockSpec((256, 128), lambda i, j: (0, j)),
            pl.BlockSpec((1, 128), lambda i, j: (0, j)),
        ],
        out_specs=[
            pl.BlockSpec((bn, 128), lambda i, j: (i, j)),
            pl.BlockSpec((bn, 128), lambda i, j: (j * nb + i, 0)),
        ],
        out_shape=[
            jax.ShapeDtypeStruct((n, 256), jnp.float32),
            jax.ShapeDtypeStruct((2 * n, 128), jnp.float32),
        ],
    )(xs_prev, agg, agg, deg, deg, Ws, Wn, b.reshape(1, 256))


def _tc_final(xs_prev, agg, deg, *, bn=1000):
    n = xs_prev.shape[0]
    nb = n // bn

    def body(xs_ref, a0_ref, a1_ref, d0_ref, d1_ref, o_ref):
        inv = 1.0 / jnp.maximum(d0_ref[:, 0:1] + d1_ref[:, 0:1], 1.0)
        agg_full = jnp.concatenate([a0_ref[...], a1_ref[...]], axis=1)
        o_ref[...] = xs_ref[...] + agg_full * inv

    return pl.pallas_call(
        body,
        grid=(nb,),
        in_specs=[
            pl.BlockSpec((bn, 256), lambda i: (i, 0)),
            pl.BlockSpec((bn, 128), lambda i: (i, 0)),
            pl.BlockSpec((bn, 128), lambda i: (nb + i, 0)),
            pl.BlockSpec((bn, 128), lambda i: (i, 0)),
            pl.BlockSpec((bn, 128), lambda i: (nb + i, 0)),
        ],
        out_specs=pl.BlockSpec((bn, 256), lambda i: (i, 0)),
        out_shape=jax.ShapeDtypeStruct((n, 256), jnp.float32),
    )(xs_prev, agg, agg, deg, deg)


_B = 40


@functools.partial(jax.jit, static_argnames=("n", "e"))
def _sc_segment_sum(y2, src_pc, dst2, z128, *, n, e):
    nch = e // _C
    per_tile = nch // _NSUB
    nbatch = per_tile // _B
    rpt = (n // _NSUB) // 8 * 8
    rem = n - _NSUB * rpt

    mesh = plsc.VectorSubcoreMesh(core_axis_name="c", subcore_axis_name="s")

    def body(y2_h, src_h, dst_h, z128_h, agg_o,
             src_v, dst_v, rows_a, rows_b, acc,
             semg_a, semg_b, sems_a, sems_b):
        c = lax.axis_index("c")
        s = lax.axis_index("s")
        r0 = s * rpt
        rr = _NSUB * rpt

        pltpu.sync_copy(z128_h.at[pl.ds(r0, rpt)], acc.at[pl.ds(r0, rpt)])

        @pl.when(s == _NSUB - 1)
        def _():
            pltpu.sync_copy(z128_h.at[pl.ds(rr, rem)], acc.at[pl.ds(rr, rem)])

        plsc.subcore_barrier()

        rows = (rows_a, rows_b)
        semg = (semg_a, semg_b)
        sems = (sems_a, sems_b)

        def batch(bi, carry):
            base = s * per_tile + bi * _B
            pltpu.sync_copy(src_h.at[pl.ds(c * nch + base, _B)], src_v)
            pltpu.sync_copy(dst_h.at[pl.ds(base, _B)], dst_v)
            gdesc = [None, None]
            sdesc = [None, None]
            gdesc[0] = pltpu.async_copy(y2_h.at[src_v.at[0]], rows[0],
                                        semg[0])
            for kk in range(_B):
                b = kk & 1
                b1 = b ^ 1
                if kk + 1 < _B:
                    if sdesc[b1] is not None:
                        sdesc[b1].wait()
                    gdesc[b1] = pltpu.async_copy(
                        y2_h.at[src_v.at[kk + 1]], rows[b1], semg[b1])
                gdesc[b].wait()
                sdesc[b] = pltpu.async_copy(rows[b], acc.at[dst_v.at[kk]],
                                            sems[b], add=True)
            sdesc[0].wait()
            sdesc[1].wait()
            return carry

        lax.fori_loop(0, nbatch, batch, 0)
        plsc.subcore_barrier()

        pltpu.sync_copy(acc.at[pl.ds(r0, rpt)],
                        agg_o.at[pl.ds(c * n + r0, rpt)])

        @pl.when(s == _NSUB - 1)
        def _():
            pltpu.sync_copy(acc.at[pl.ds(rr, rem)],
                            agg_o.at[pl.ds(c * n + rr, rem)])

    fn = pl.kernel(
        body,
        out_type=jax.ShapeDtypeStruct((2 * n, 128), jnp.float32),
        mesh=mesh,
        scratch_types=[
            pltpu.VMEM((_B, _C), jnp.int32),
            pltpu.VMEM((_B, _C), jnp.int32),
            pltpu.VMEM((_C, 128), jnp.float32),
            pltpu.VMEM((_C, 128), jnp.float32),
            pltpu.VMEM_SHARED((n, 128), jnp.float32),
            pltpu.SemaphoreType.DMA,
            pltpu.SemaphoreType.DMA,
            pltpu.SemaphoreType.DMA,
            pltpu.SemaphoreType.DMA,
        ],
    )
    return fn(y2, src_pc, dst2, z128)


@functools.partial(jax.jit, static_argnames=("n", "e"))
def _sc_degree(dst2, ones128, z128, *, n, e):
    nch = e // _C
    half = nch // 2
    per_tile = half // _NSUB
    db = 8
    nbatch = per_tile // db
    rpt = (n // _NSUB) // 8 * 8
    rem = n - _NSUB * rpt

    mesh = plsc.VectorSubcoreMesh(core_axis_name="c", subcore_axis_name="s")

    def body(dst_h, ones_h, z128_h, deg_o, dst_v, ones_v, accd, sem):
        c = lax.axis_index("c")
        s = lax.axis_index("s")
        r0 = s * rpt
        rr = _NSUB * rpt

        pltpu.sync_copy(z128_h.at[pl.ds(r0, rpt)], accd.at[pl.ds(r0, rpt)])
        pltpu.sync_copy(ones_h, ones_v)

        @pl.when(s == _NSUB - 1)
        def _():
            pltpu.sync_copy(z128_h.at[pl.ds(rr, rem)], accd.at[pl.ds(rr, rem)])

        plsc.subcore_barrier()

        def batch(bi, carry):
            base = c * half + s * per_tile + bi * db
            pltpu.sync_copy(dst_h.at[pl.ds(base, db)], dst_v)
            for kk in range(db):
                pltpu.async_copy(ones_v, accd.at[dst_v.at[kk]], sem, add=True)
            for kk in range(db):
                pltpu.make_async_copy(ones_v, accd.at[dst_v.at[kk]],
                                      sem).wait()
            return carry

        lax.fori_loop(0, nbatch, batch, 0)
        plsc.subcore_barrier()
        pltpu.sync_copy(accd.at[pl.ds(r0, rpt)],
                        deg_o.at[pl.ds(c * n + r0, rpt)])

        @pl.when(s == _NSUB - 1)
        def _():
            pltpu.sync_copy(accd.at[pl.ds(rr, rem)],
                            deg_o.at[pl.ds(c * n + rr, rem)])

    fn = pl.kernel(
        body,
        out_type=jax.ShapeDtypeStruct((2 * n, 128), jnp.float32),
        mesh=mesh,
        scratch_types=[
            pltpu.VMEM((8, _C), jnp.int32),
            pltpu.VMEM((_C, 128), jnp.float32),
            pltpu.VMEM_SHARED((n, 128), jnp.float32),
            pltpu.SemaphoreType.DMA,
        ],
    )
    return fn(dst2, ones128, z128)


def kernel(x, edge_index, Ws0, Wn0, b0, Ws1, Wn1, b1, Ws2, Wn2, b2):
    n = x.shape[0]
    e = edge_index.shape[1]
    src = edge_index[0].astype(jnp.int32)
    dst = edge_index[1].astype(jnp.int32)

    src_pc = jnp.concatenate([src, src + n]).reshape(2 * (e // _C), _C)
    dst2 = dst.reshape(e // _C, _C)
    ones128 = jnp.ones((_C, 128), jnp.float32)
    z128 = jnp.zeros((n, 128), jnp.float32)

    deg = _sc_degree(dst2, ones128, z128, n=n, e=e)
    xs0, y0 = _tc_mm0(x, Ws0, Wn0, b0)
    agg0 = _sc_segment_sum(y0, src_pc, dst2, z128, n=n, e=e)
    xs1, y1 = _tc_combine_mm(xs0, agg0, deg, Ws1, Wn1, b1)
    agg1 = _sc_segment_sum(y1, src_pc, dst2, z128, n=n, e=e)
    xs2, y2 = _tc_combine_mm(xs1, agg1, deg, Ws2, Wn2, b2)
    agg2 = _sc_segment_sum(y2, src_pc, dst2, z128, n=n, e=e)
    return _tc_final(xs2, agg2, deg)
